# baseline (speedup 1.0000x reference)
import functools

import jax
import jax.numpy as jnp
from jax import lax
from jax.experimental import pallas as pl
from jax.experimental.pallas import tpu as pltpu
from jax.experimental.pallas import tpu_sc as plsc

NTOKEN = 100000
EMB = 16
BATCH = 1024
CTX = 20

NC = 2
NS = 16
NW = NC * NS
IDX_PER_W = BATCH * CTX // NW
ROWS_PER_W = BATCH // NW
CHUNK = 128

TN = 1024


def _sc_pool_body(idx_hbm, table_hbm, out_hbm, idx_v, rows_v, hid_v, sem):
    wid = lax.axis_index("s") * NC + lax.axis_index("c")
    base = wid * IDX_PER_W
    pltpu.sync_copy(idx_hbm.at[pl.ds(base, IDX_PER_W)], idx_v)
    copies = []
    for k in range(IDX_PER_W // CHUNK):
        copies.append(
            pltpu.async_copy(
                table_hbm.at[idx_v.at[pl.ds(k * CHUNK, CHUNK)]],
                rows_v.at[pl.ds(k * CHUNK, CHUNK)],
                sem,
            )
        )
    for c in copies:
        c.wait()

    @pl.loop(0, ROWS_PER_W)
    def _(e):
        r0 = e * CTX
        acc = rows_v[pl.ds(r0, 1), :]
        for c in range(1, CTX):
            acc = acc + rows_v[pl.ds(r0 + c, 1), :]
        hid_v[pl.ds(e, 1), :] = acc * (1.0 / CTX)

    pltpu.sync_copy(hid_v, out_hbm.at[pl.ds(wid * ROWS_PER_W, ROWS_PER_W)])


def _sc_hidden(idx_flat, emb_table):
    mesh = plsc.VectorSubcoreMesh(core_axis_name="c", subcore_axis_name="s")
    k = pl.kernel(
        _sc_pool_body,
        out_type=jax.ShapeDtypeStruct((BATCH, EMB), jnp.float32),
        mesh=mesh,
        compiler_params=pltpu.CompilerParams(use_tc_tiling_on_sc=False),
        scratch_types=[
            pltpu.VMEM((IDX_PER_W,), jnp.int32),
            pltpu.VMEM((IDX_PER_W, EMB), jnp.float32),
            pltpu.VMEM((ROWS_PER_W, EMB), jnp.float32),
            pltpu.SemaphoreType.DMA,
        ],
    )
    return k(idx_flat, emb_table)


NOB = 8
NGROUPS = 12
N_FULL = 97
TAIL = NTOKEN - N_FULL * TN


def _mm_body(h_ref, wt_ref, b_ref, o_hbm, *scratch):
    obufs = scratch[0:NOB]
    tbuf = scratch[NOB]
    osems = scratch[NOB + 1 : 2 * NOB + 1]
    tsem = scratch[2 * NOB + 1]
    h = h_ref[...].astype(jnp.bfloat16)

    def tile_out(idx, width):
        wt = wt_ref[:, pl.ds(idx, width)]
        acc = jax.lax.dot_general(
            h, wt, (((1,), (0,)), ((), ())), preferred_element_type=jnp.float32
        )
        return acc + b_ref[:, pl.ds(idx, width)]

    def ocopy(k, idx):
        return pltpu.make_async_copy(
            obufs[k], o_hbm.at[:, pl.ds(idx, TN)], osems[k]
        )

    def ocopy_start_chunks(k, idx):
        for r in range(0, BATCH, 8):
            pltpu.make_async_copy(
                obufs[k].at[pl.ds(r, 8), :],
                o_hbm.at[pl.ds(r, 8), pl.ds(idx, TN)],
                osems[k],
            ).start()

    @pl.loop(0, NGROUPS)
    def _(g):
        for k in range(NOB):
            idx = pl.multiple_of((g * NOB + k) * TN, TN)

            @pl.when(g > 0)
            def _():
                ocopy(k, idx).wait()

            obufs[k][...] = tile_out(idx, TN)
            ocopy_start_chunks(k, idx)

    ocopy(0, 0).wait()
    obufs[0][...] = tile_out(96 * TN, TN)
    ocopy_start_chunks(0, 96 * TN)

    tbuf[...] = tile_out(N_FULL * TN, TAIL)
    for r in range(0, BATCH, 8):
        pltpu.make_async_copy(
            tbuf.at[pl.ds(r, 8), :],
            o_hbm.at[pl.ds(r, 8), pl.ds(N_FULL * TN, TAIL)],
            tsem,
        ).start()
    tcopy = pltpu.make_async_copy(
        tbuf, o_hbm.at[:, pl.ds(N_FULL * TN, TAIL)], tsem
    )

    for k in range(1, NOB):
        ocopy(k, 0).wait()
    ocopy(0, 0).wait()
    tcopy.wait()


def _decode(hidden, wt_bf16, b_row):
    return pl.pallas_call(
        _mm_body,
        in_specs=[
            pl.BlockSpec(memory_space=pltpu.MemorySpace.VMEM),
            pl.BlockSpec(memory_space=pltpu.MemorySpace.VMEM),
            pl.BlockSpec(memory_space=pltpu.MemorySpace.VMEM),
        ],
        out_specs=pl.BlockSpec(memory_space=pltpu.MemorySpace.HBM),
        out_shape=jax.ShapeDtypeStruct((BATCH, NTOKEN), jnp.float32),
        scratch_shapes=(
            [pltpu.VMEM((BATCH, TN), jnp.float32) for _ in range(NOB)]
            + [pltpu.VMEM((BATCH, TAIL), jnp.float32)]
            + [pltpu.SemaphoreType.DMA for _ in range(NOB + 1)]
        ),
        compiler_params=pltpu.CompilerParams(
            vmem_limit_bytes=100 * 1024 * 1024,
        ),
    )(hidden, wt_bf16, b_row)


def kernel(input, emb_table, W, b):
    idx_flat = input.astype(jnp.int32).reshape(-1)
    hidden = _sc_hidden(idx_flat, emb_table)
    wt_bf16 = W.T.astype(jnp.bfloat16)
    b_row = b.reshape(1, NTOKEN)
    return _decode(hidden, wt_bf16, b_row)


# speedup vs baseline: 1.0047x; 1.0047x over previous
import functools

import jax
import jax.numpy as jnp
from jax import lax
from jax.experimental import pallas as pl
from jax.experimental.pallas import tpu as pltpu
from jax.experimental.pallas import tpu_sc as plsc

NTOKEN = 100000
EMB = 16
BATCH = 1024
CTX = 20

NC = 2
NS = 16
NW = NC * NS
IDX_PER_W = BATCH * CTX // NW
ROWS_PER_W = BATCH // NW
CHUNK = 128

TN = 1024


def _sc_pool_body(idx_hbm, table_hbm, out_hbm, idx_v, rows_v, hid_v, sem):
    wid = lax.axis_index("s") * NC + lax.axis_index("c")
    base = wid * IDX_PER_W
    pltpu.sync_copy(idx_hbm.at[pl.ds(base, IDX_PER_W)], idx_v)
    copies = []
    for k in range(IDX_PER_W // CHUNK):
        copies.append(
            pltpu.async_copy(
                table_hbm.at[idx_v.at[pl.ds(k * CHUNK, CHUNK)]],
                rows_v.at[pl.ds(k * CHUNK, CHUNK)],
                sem,
            )
        )
    for c in copies:
        c.wait()

    @pl.loop(0, ROWS_PER_W)
    def _(e):
        r0 = e * CTX
        acc = rows_v[pl.ds(r0, 1), :]
        for c in range(1, CTX):
            acc = acc + rows_v[pl.ds(r0 + c, 1), :]
        hid_v[pl.ds(e, 1), :] = acc * (1.0 / CTX)

    pltpu.sync_copy(hid_v, out_hbm.at[pl.ds(wid * ROWS_PER_W, ROWS_PER_W)])


def _sc_hidden(idx_flat, emb_table):
    mesh = plsc.VectorSubcoreMesh(core_axis_name="c", subcore_axis_name="s")
    k = pl.kernel(
        _sc_pool_body,
        out_type=jax.ShapeDtypeStruct((BATCH, EMB), jnp.float32),
        mesh=mesh,
        compiler_params=pltpu.CompilerParams(use_tc_tiling_on_sc=False),
        scratch_types=[
            pltpu.VMEM((IDX_PER_W,), jnp.int32),
            pltpu.VMEM((IDX_PER_W, EMB), jnp.float32),
            pltpu.VMEM((ROWS_PER_W, EMB), jnp.float32),
            pltpu.SemaphoreType.DMA,
        ],
    )
    return k(idx_flat, emb_table)


NOB = 8
NGROUPS = 12
N_FULL = 97
TAIL = NTOKEN - N_FULL * TN


def _mm_body(h_ref, wt_ref, b_ref, o_hbm, *scratch):
    obufs = scratch[0:NOB]
    tbuf = scratch[NOB]
    osems = scratch[NOB + 1 : NOB + 1 + 4 * NOB]
    tsem = scratch[NOB + 1 + 4 * NOB]
    h = h_ref[...].astype(jnp.bfloat16)

    def tile_out(idx, width):
        wt = wt_ref[:, pl.ds(idx, width)]
        acc = jax.lax.dot_general(
            h, wt, (((1,), (0,)), ((), ())), preferred_element_type=jnp.float32
        )
        return acc + b_ref[:, pl.ds(idx, width)]

    QROWS = BATCH // 4

    def oquarter(k, q, idx):
        return pltpu.make_async_copy(
            obufs[k].at[pl.ds(q * QROWS, QROWS), :],
            o_hbm.at[pl.ds(q * QROWS, QROWS), pl.ds(idx, TN)],
            osems[4 * k + q],
        )

    def ocopy_start_chunks(k, idx):
        for q in range(4):
            oquarter(k, q, idx).start()

    def ocopy(k, idx):
        class _Waiter:
            def wait(self):
                for q in range(4):
                    oquarter(k, q, idx).wait()

        return _Waiter()

    @pl.loop(0, NGROUPS)
    def _(g):
        for k in range(NOB):
            idx = pl.multiple_of((g * NOB + k) * TN, TN)

            @pl.when(g > 0)
            def _():
                ocopy(k, idx).wait()

            obufs[k][...] = tile_out(idx, TN)
            ocopy_start_chunks(k, idx)

    ocopy(0, 0).wait()
    obufs[0][...] = tile_out(96 * TN, TN)
    ocopy_start_chunks(0, 96 * TN)

    tbuf[...] = tile_out(N_FULL * TN, TAIL)
    for r in range(0, BATCH, 8):
        pltpu.make_async_copy(
            tbuf.at[pl.ds(r, 8), :],
            o_hbm.at[pl.ds(r, 8), pl.ds(N_FULL * TN, TAIL)],
            tsem,
        ).start()
    tcopy = pltpu.make_async_copy(
        tbuf, o_hbm.at[:, pl.ds(N_FULL * TN, TAIL)], tsem
    )

    for k in range(1, NOB):
        ocopy(k, 0).wait()
    ocopy(0, 0).wait()
    tcopy.wait()


def _decode(hidden, wt_bf16, b_row):
    return pl.pallas_call(
        _mm_body,
        in_specs=[
            pl.BlockSpec(memory_space=pltpu.MemorySpace.VMEM),
            pl.BlockSpec(memory_space=pltpu.MemorySpace.VMEM),
            pl.BlockSpec(memory_space=pltpu.MemorySpace.VMEM),
        ],
        out_specs=pl.BlockSpec(memory_space=pltpu.MemorySpace.HBM),
        out_shape=jax.ShapeDtypeStruct((BATCH, NTOKEN), jnp.float32),
        scratch_shapes=(
            [pltpu.VMEM((BATCH, TN), jnp.float32) for _ in range(NOB)]
            + [pltpu.VMEM((BATCH, TAIL), jnp.float32)]
            + [pltpu.SemaphoreType.DMA for _ in range(4 * NOB + 1)]
        ),
        compiler_params=pltpu.CompilerParams(
            vmem_limit_bytes=100 * 1024 * 1024,
        ),
    )(hidden, wt_bf16, b_row)


def kernel(input, emb_table, W, b):
    idx_flat = input.astype(jnp.int32).reshape(-1)
    hidden = _sc_hidden(idx_flat, emb_table)
    wt_bf16 = W.T.astype(jnp.bfloat16)
    b_row = b.reshape(1, NTOKEN)
    return _decode(hidden, wt_bf16, b_row)


# speedup vs baseline: 2.7530x; 2.7401x over previous
import functools

import jax
import jax.numpy as jnp
from jax import lax
from jax.experimental import pallas as pl
from jax.experimental.pallas import tpu as pltpu
from jax.experimental.pallas import tpu_sc as plsc

NTOKEN = 100000
EMB = 16
BATCH = 1024
CTX = 20

NC = 2
NS = 16
NW = NC * NS
IDX_PER_W = BATCH * CTX // NW
ROWS_PER_W = BATCH // NW
CHUNK = 128

TN = 1024
NOB = 8
NGROUPS = 12
N_FULL = 97
TAIL = NTOKEN - N_FULL * TN


def _sc_pool_body(idx_hbm, table_hbm, out_hbm, idx_v, rows_v, hid_v, sem):
    wid = lax.axis_index("s") * NC + lax.axis_index("c")
    base = wid * IDX_PER_W
    pltpu.sync_copy(idx_hbm.at[pl.ds(base, IDX_PER_W)], idx_v)
    copies = []
    for k in range(IDX_PER_W // CHUNK):
        copies.append(
            pltpu.async_copy(
                table_hbm.at[idx_v.at[pl.ds(k * CHUNK, CHUNK)]],
                rows_v.at[pl.ds(k * CHUNK, CHUNK)],
                sem,
            )
        )
    for c in copies:
        c.wait()

    @pl.loop(0, ROWS_PER_W)
    def _(e):
        r0 = e * CTX
        acc = rows_v[pl.ds(r0, 1), :]
        for c in range(1, CTX):
            acc = acc + rows_v[pl.ds(r0 + c, 1), :]
        hid_v[pl.ds(e, 1), :] = acc * (1.0 / CTX)

    pltpu.sync_copy(hid_v, out_hbm.at[pl.ds(wid * ROWS_PER_W, ROWS_PER_W)])


def _sc_hidden(idx_flat, emb_table):
    mesh = plsc.VectorSubcoreMesh(core_axis_name="c", subcore_axis_name="s")
    k = pl.kernel(
        _sc_pool_body,
        out_type=jax.ShapeDtypeStruct((BATCH, EMB), jnp.float32),
        mesh=mesh,
        compiler_params=pltpu.CompilerParams(use_tc_tiling_on_sc=False),
        scratch_types=[
            pltpu.VMEM((IDX_PER_W,), jnp.int32),
            pltpu.VMEM((IDX_PER_W, EMB), jnp.float32),
            pltpu.VMEM((ROWS_PER_W, EMB), jnp.float32),
            pltpu.SemaphoreType.DMA,
        ],
    )
    return k(idx_flat, emb_table)


def _mm_body(h_ref, wt_ref, b_ref, o_hbm, *scratch):
    obufs = scratch[0:NOB]
    tbuf = scratch[NOB]
    osems = scratch[NOB + 1 : NOB + 1 + 4 * NOB]
    tsem = scratch[NOB + 1 + 4 * NOB]
    ht = jnp.transpose(h_ref[...]).astype(jnp.bfloat16)

    def tile_out(idx, width):
        wt = wt_ref[:, pl.ds(idx, width)]
        acc = jax.lax.dot_general(
            wt, ht, (((0,), (0,)), ((), ())), preferred_element_type=jnp.float32
        )
        bcol = jnp.transpose(b_ref[:, pl.ds(idx, width)])
        return acc + bcol

    QROWS = TN // 4

    def oquarter(k, q, idx):
        return pltpu.make_async_copy(
            obufs[k].at[pl.ds(q * QROWS, QROWS), :],
            o_hbm.at[pl.ds(idx + q * QROWS, QROWS), :],
            osems[4 * k + q],
        )

    def ocopy_start_chunks(k, idx):
        for q in range(4):
            oquarter(k, q, idx).start()

    def owait(k, idx):
        for q in range(4):
            oquarter(k, q, idx).wait()

    @pl.loop(0, NGROUPS)
    def _(g):
        for k in range(NOB):
            idx = pl.multiple_of((g * NOB + k) * TN, TN)

            @pl.when(g > 0)
            def _():
                owait(k, idx)

            obufs[k][...] = tile_out(idx, TN)
            ocopy_start_chunks(k, idx)

    owait(0, 0)
    obufs[0][...] = tile_out(96 * TN, TN)
    ocopy_start_chunks(0, 96 * TN)

    tbuf[...] = tile_out(N_FULL * TN, TAIL)
    tcopy = pltpu.make_async_copy(
        tbuf, o_hbm.at[pl.ds(N_FULL * TN, TAIL), :], tsem
    )
    tcopy.start()

    for k in range(1, NOB):
        owait(k, 0)
    owait(0, 0)
    tcopy.wait()


def _decode_t(hidden, wt_bf16, b_row):
    return pl.pallas_call(
        _mm_body,
        in_specs=[
            pl.BlockSpec(memory_space=pltpu.MemorySpace.VMEM),
            pl.BlockSpec(memory_space=pltpu.MemorySpace.VMEM),
            pl.BlockSpec(memory_space=pltpu.MemorySpace.VMEM),
        ],
        out_specs=pl.BlockSpec(memory_space=pltpu.MemorySpace.HBM),
        out_shape=jax.ShapeDtypeStruct((NTOKEN, BATCH), jnp.float32),
        scratch_shapes=(
            [pltpu.VMEM((TN, BATCH), jnp.float32) for _ in range(NOB)]
            + [pltpu.VMEM((TAIL, BATCH), jnp.float32)]
            + [pltpu.SemaphoreType.DMA for _ in range(4 * NOB + 1)]
        ),
        compiler_params=pltpu.CompilerParams(
            vmem_limit_bytes=100 * 1024 * 1024,
        ),
    )(hidden, wt_bf16, b_row)


def kernel(input, emb_table, W, b):
    idx_flat = input.astype(jnp.int32).reshape(-1)
    hidden = _sc_hidden(idx_flat, emb_table)
    wt_bf16 = W.T.astype(jnp.bfloat16)
    b_row = b.reshape(1, NTOKEN)
    return _decode_t(hidden, wt_bf16, b_row).T


# speedup vs baseline: 2.7639x; 1.0040x over previous
import functools

import jax
import jax.numpy as jnp
from jax import lax
from jax.experimental import pallas as pl
from jax.experimental.pallas import tpu as pltpu
from jax.experimental.pallas import tpu_sc as plsc

NTOKEN = 100000
EMB = 16
BATCH = 1024
CTX = 20

NC = 2
NS = 16
NW = NC * NS
IDX_PER_W = BATCH * CTX // NW
ROWS_PER_W = BATCH // NW
CHUNK = 128

TN = 1024
NOB = 8
NGROUPS = 12
N_FULL = 97
TAIL = NTOKEN - N_FULL * TN


def _sc_pool_body(idx_hbm, table_hbm, out_hbm, idx_v, rows_v, hid_v, sem):
    wid = lax.axis_index("s") * NC + lax.axis_index("c")
    base = wid * IDX_PER_W
    pltpu.sync_copy(idx_hbm.at[pl.ds(base, IDX_PER_W)], idx_v)
    copies = []
    for k in range(IDX_PER_W // CHUNK):
        copies.append(
            pltpu.async_copy(
                table_hbm.at[idx_v.at[pl.ds(k * CHUNK, CHUNK)]],
                rows_v.at[pl.ds(k * CHUNK, CHUNK)],
                sem,
            )
        )
    for c in copies:
        c.wait()

    @pl.loop(0, ROWS_PER_W)
    def _(e):
        r0 = e * CTX
        acc = rows_v[pl.ds(r0, 1), :]
        for c in range(1, CTX):
            acc = acc + rows_v[pl.ds(r0 + c, 1), :]
        hid_v[pl.ds(e, 1), :] = acc * (1.0 / CTX)

    pltpu.sync_copy(hid_v, out_hbm.at[pl.ds(wid * ROWS_PER_W, ROWS_PER_W)])


def _sc_hidden(idx_flat, emb_table):
    mesh = plsc.VectorSubcoreMesh(core_axis_name="c", subcore_axis_name="s")
    k = pl.kernel(
        _sc_pool_body,
        out_type=jax.ShapeDtypeStruct((BATCH, EMB), jnp.float32),
        mesh=mesh,
        compiler_params=pltpu.CompilerParams(use_tc_tiling_on_sc=False),
        scratch_types=[
            pltpu.VMEM((IDX_PER_W,), jnp.int32),
            pltpu.VMEM((IDX_PER_W, EMB), jnp.float32),
            pltpu.VMEM((ROWS_PER_W, EMB), jnp.float32),
            pltpu.SemaphoreType.DMA,
        ],
    )
    return k(idx_flat, emb_table)


def _mm_body(h_ref, wt_ref, b_ref, o_hbm, *scratch):
    obufs = scratch[0:NOB]
    tbuf = scratch[NOB]
    osems = scratch[NOB + 1 : NOB + 1 + 4 * NOB]
    tsem = scratch[NOB + 1 + 4 * NOB]
    ht = jnp.transpose(h_ref[...]).astype(jnp.bfloat16)

    def tile_out(idx, width):
        wt = wt_ref[:, pl.ds(idx, width)]
        acc = jax.lax.dot_general(
            wt, ht, (((0,), (0,)), ((), ())), preferred_element_type=jnp.float32
        )
        bcol = jnp.transpose(b_ref[:, pl.ds(idx, width)])
        return acc + bcol

    NSPLIT = 2
    QROWS = TN // NSPLIT

    def oquarter(k, q, idx):
        return pltpu.make_async_copy(
            obufs[k].at[pl.ds(q * QROWS, QROWS), :],
            o_hbm.at[pl.ds(idx + q * QROWS, QROWS), :],
            osems[4 * k + q],
        )

    def ocopy_start_chunks(k, idx):
        for q in range(NSPLIT):
            oquarter(k, q, idx).start()

    def owait(k, idx):
        for q in range(NSPLIT):
            oquarter(k, q, idx).wait()

    @pl.loop(0, NGROUPS)
    def _(g):
        for k in range(NOB):
            idx = pl.multiple_of((g * NOB + k) * TN, TN)

            @pl.when(g > 0)
            def _():
                owait(k, idx)

            obufs[k][...] = tile_out(idx, TN)
            ocopy_start_chunks(k, idx)

    owait(0, 0)
    obufs[0][...] = tile_out(96 * TN, TN)
    ocopy_start_chunks(0, 96 * TN)

    tbuf[...] = tile_out(N_FULL * TN, TAIL)
    tcopy = pltpu.make_async_copy(
        tbuf, o_hbm.at[pl.ds(N_FULL * TN, TAIL), :], tsem
    )
    tcopy.start()

    for k in range(1, NOB):
        owait(k, 0)
    owait(0, 0)
    tcopy.wait()


def _decode_t(hidden, wt_bf16, b_row):
    return pl.pallas_call(
        _mm_body,
        in_specs=[
            pl.BlockSpec(memory_space=pltpu.MemorySpace.VMEM),
            pl.BlockSpec(memory_space=pltpu.MemorySpace.VMEM),
            pl.BlockSpec(memory_space=pltpu.MemorySpace.VMEM),
        ],
        out_specs=pl.BlockSpec(memory_space=pltpu.MemorySpace.HBM),
        out_shape=jax.ShapeDtypeStruct((NTOKEN, BATCH), jnp.float32),
        scratch_shapes=(
            [pltpu.VMEM((TN, BATCH), jnp.float32) for _ in range(NOB)]
            + [pltpu.VMEM((TAIL, BATCH), jnp.float32)]
            + [pltpu.SemaphoreType.DMA for _ in range(4 * NOB + 1)]
        ),
        compiler_params=pltpu.CompilerParams(
            vmem_limit_bytes=100 * 1024 * 1024,
        ),
    )(hidden, wt_bf16, b_row)


def kernel(input, emb_table, W, b):
    idx_flat = input.astype(jnp.int32).reshape(-1)
    hidden = _sc_hidden(idx_flat, emb_table)
    wt_bf16 = W.T.astype(jnp.bfloat16)
    b_row = b.reshape(1, NTOKEN)
    return _decode_t(hidden, wt_bf16, b_row).T
